# baseline (speedup 1.0000x reference)
import functools

import jax
import jax.numpy as jnp
from jax import lax
from jax.experimental import pallas as pl
from jax.experimental.pallas import tpu as pltpu
from jax.experimental.pallas import tpu_sc as plsc

N_NODES = 10000
NP = 10240
G = 128
E_RAW = 160000
E_LOOP = E_RAW + N_NODES
CH = 64
CHA = 128
E2 = 172032
NC, NS = 2, 16
EH = E2 // NC
ET = EH // NS
ETA = E2 // NS
RT = NP // NS
NB = 512
NI = NP // NB

_f32 = jnp.float32


def _pack_i32(x):
    lo = lax.bitcast_convert_type(
        x[:, :128].astype(jnp.bfloat16), jnp.uint16).astype(jnp.uint32)
    hi = lax.bitcast_convert_type(
        x[:, 128:].astype(jnp.bfloat16), jnp.uint16).astype(jnp.uint32)
    return lax.bitcast_convert_type(lo | (hi << 16), jnp.int32)


def _dual_mm(h4, Ws, Wd):
    KB = h4.shape[0]

    def body(a_ref, ws_ref, wd_ref, xle_ref, xlo_ref, xlp_ref, xrp_ref,
             acc_l, acc_r):
        k = pl.program_id(2)

        @pl.when(k == 0)
        def _():
            acc_l[...] = jnp.zeros_like(acc_l)
            acc_r[...] = jnp.zeros_like(acc_r)

        a = a_ref[0]
        acc_l[...] += jnp.dot(a, ws_ref[...], preferred_element_type=_f32)
        acc_r[...] += jnp.dot(a, wd_ref[...], preferred_element_type=_f32)

        @pl.when(k == KB - 1)
        def _():
            al = acc_l[...]
            xle_ref[0] = al[:, :128]
            xlo_ref[0] = al[:, 128:]
            xlp_ref[0] = _pack_i32(al)
            xrp_ref[0] = _pack_i32(acc_r[...])

    two = jax.ShapeDtypeStruct((2, NP, 128), _f32)
    twoi = jax.ShapeDtypeStruct((2, NP, 128), jnp.int32)
    return pl.pallas_call(
        body,
        grid=(NI, 2, KB),
        in_specs=[
            pl.BlockSpec((1, NB, 128), lambda i, j, k: (k, i, 0)),
            pl.BlockSpec((128, 256), lambda i, j, k: (k, j)),
            pl.BlockSpec((128, 256), lambda i, j, k: (k, j)),
        ],
        out_specs=[
            pl.BlockSpec((1, NB, 128), lambda i, j, k: (j, i, 0)),
            pl.BlockSpec((1, NB, 128), lambda i, j, k: (j, i, 0)),
            pl.BlockSpec((1, NB, 128), lambda i, j, k: (j, i, 0)),
            pl.BlockSpec((1, NB, 128), lambda i, j, k: (j, i, 0)),
        ],
        out_shape=[two, two, twoi, twoi],
        scratch_shapes=[pltpu.VMEM((NB, 256), _f32),
                        pltpu.VMEM((NB, 256), _f32)],
    )(h4, Ws, Wd)


def _normalize(num4, dp32, b4):
    def body(num_ref, dp_ref, b_ref, out_ref):
        denom = jnp.sum(dp_ref[...], axis=0) + 1e-16
        h = num_ref[0] / denom[:, None] + b_ref[0]
        out_ref[0] = jnp.maximum(h, 0.0)

    return pl.pallas_call(
        body,
        grid=(4, NI),
        in_specs=[
            pl.BlockSpec((1, NB, 128), lambda kb, i: (kb, i, 0)),
            pl.BlockSpec((32, NB), lambda kb, i: (0, i)),
            pl.BlockSpec((1, 1, 128), lambda kb, i: (kb, 0, 0)),
        ],
        out_specs=pl.BlockSpec((1, NB, 128), lambda kb, i: (kb, i, 0)),
        out_shape=jax.ShapeDtypeStruct((4, NP, 128), _f32),
    )(num4, dp32, b4)


def _pool_mlp(h4, batch_p, W1, bm1, W2, bm2):
    def body(h_ref, bt_ref, w1_ref, b1_ref, w2_ref, b2_ref, out_ref,
             sums_ref, cnt_ref):
        i = pl.program_id(0)

        @pl.when(i == 0)
        def _():
            sums_ref[...] = jnp.zeros_like(sums_ref)
            cnt_ref[...] = jnp.zeros_like(cnt_ref)

        gid = lax.broadcasted_iota(jnp.int32, (G, NB), 0)
        oh = (gid == bt_ref[...][None, :]).astype(_f32)
        for kb in range(4):
            sums_ref[:, kb * 128:(kb + 1) * 128] += jnp.dot(
                oh, h_ref[kb], preferred_element_type=_f32)
        cnt_ref[...] += jnp.dot(oh, jnp.ones((NB, 128), _f32),
                                preferred_element_type=_f32)

        @pl.when(i == NI - 1)
        def _():
            cnt = jnp.maximum(cnt_ref[:, 0:1], 1.0)
            pooled = sums_ref[...] / cnt
            z = jnp.maximum(
                jnp.dot(pooled, w1_ref[...], preferred_element_type=_f32)
                + b1_ref[...], 0.0)
            out_ref[...] = (jnp.dot(z, w2_ref[...],
                                    preferred_element_type=_f32)
                            + b2_ref[...])

    return pl.pallas_call(
        body,
        grid=(NI,),
        in_specs=[
            pl.BlockSpec((4, NB, 128), lambda i: (0, i, 0)),
            pl.BlockSpec((NB,), lambda i: (i,)),
            pl.BlockSpec((512, 512), lambda i: (0, 0)),
            pl.BlockSpec((1, 512), lambda i: (0, 0)),
            pl.BlockSpec((512, 256), lambda i: (0, 0)),
            pl.BlockSpec((1, 256), lambda i: (0, 0)),
        ],
        out_specs=pl.BlockSpec((G, 256), lambda i: (0, 0)),
        out_shape=jax.ShapeDtypeStruct((G, 256), _f32),
        scratch_shapes=[
            pltpu.VMEM((G, 512), _f32),
            pltpu.VMEM((G, 128), _f32),
        ],
    )(h4, batch_p, W1, bm1, W2, bm2)


def _phase_e(xl, xr, src, dst, att4):
    mesh = plsc.VectorSubcoreMesh(core_axis_name="c", subcore_axis_name="s")
    bf16 = jnp.bfloat16

    @functools.partial(
        pl.kernel,
        out_type=(jax.ShapeDtypeStruct((E2,), _f32),
                  jax.ShapeDtypeStruct((NC * NS, NP), _f32)),
        mesh=mesh,
        scratch_types=[
            pltpu.VMEM((ET,), jnp.int32),
            pltpu.VMEM((ET,), jnp.int32),
            pltpu.VMEM((2, 2, CH, 128), jnp.int32),
            pltpu.VMEM((2, 2, CH, 128), jnp.int32),
            pltpu.VMEM((ET,), _f32),
            pltpu.VMEM((CH * 16,), _f32),
            pltpu.VMEM((NP,), _f32),
            pltpu.VMEM((256,), jnp.int32),
            pltpu.SemaphoreType.DMA,
        ],
        compiler_params=pltpu.CompilerParams(needs_layout_passes=False),
    )
    def run(xlp_h, xrp_h, src_h, dst_h, att_h,
            w_h, dp_h, si, di, xlb, xrb, wa, tmpb, dacc, attb, sem):
        c = lax.axis_index("c")
        s = lax.axis_index("s")
        wid = c * NS + s
        tbase = c * EH + s * ET
        pltpu.sync_copy(src_h.at[pl.ds(tbase, ET)], si)
        pltpu.sync_copy(dst_h.at[pl.ds(tbase, ET)], di)
        pltpu.sync_copy(att_h, attb)

        def zero(i, _):
            dacc[pl.ds(i * 16, 16)] = jnp.zeros((16,), _f32)
            return 0

        lax.fori_loop(0, NP // 16, zero, 0)
        att_regs = [plsc.bitcast(attb[pl.ds(128 * tb + 16 * j, 16)], bf16)
                    for tb in range(2) for j in range(8)]

        def fire(t, buf):
            isl = si.at[pl.ds(t * CH, CH)]
            idl = di.at[pl.ds(t * CH, CH)]
            for tb in range(2):
                pltpu.async_copy(xlp_h.at[tb].at[isl], xlb.at[buf, tb], sem)
            for tb in range(2):
                pltpu.async_copy(xrp_h.at[tb].at[idl], xrb.at[buf, tb], sem)

        def drain(buf):
            for tb in range(2):
                pltpu.make_async_copy(xlp_h.at[tb].at[pl.ds(0, CH)],
                                      xlb.at[buf, tb], sem).wait()
            for tb in range(2):
                pltpu.make_async_copy(xrp_h.at[tb].at[pl.ds(0, CH)],
                                      xrb.at[buf, tb], sem).wait()

        fire(0, 0)

        def chunk(t, _):
            buf = lax.rem(t, 2)
            drain(buf)

            @pl.when(t + 1 < ET // CH)
            def _():
                fire(t + 1, 1 - buf)

            def edge(e, _):
                acc = jnp.zeros((16,), _f32)
                for tb in range(2):
                    for j in range(8):
                        a = plsc.bitcast(
                            xlb[buf, tb, e, pl.ds(16 * j, 16)], bf16)
                        bb = plsc.bitcast(
                            xrb[buf, tb, e, pl.ds(16 * j, 16)], bf16)
                        h = a + bb
                        r = jnp.maximum(h, 0) + bf16(0.2) * jnp.minimum(h, 0)
                        m = r * att_regs[tb * 8 + j]
                        u0, u1 = plsc.unpack(
                            m, format=plsc.PackFormat.INTERLEAVED)
                        acc = acc + (u0 + u1)
                tmpb[pl.ds(e * 16, 16)] = acc
                return 0

            lax.fori_loop(0, CH, edge, 0, unroll=4)
            lane = lax.broadcasted_iota(jnp.int32, (16,), 0)
            for q in range(CH // 16):
                rowi = (q * 16 + lane) * 16
                sv = jnp.zeros((16,), _f32)
                for k in range(16):
                    sv = sv + plsc.load_gather(tmpb, [rowi + k])
                w16 = jnp.exp(sv)
                wa[pl.ds(t * CH + q * 16, 16)] = w16
                plsc.addupdate_scatter(
                    dacc, [di[pl.ds(t * CH + q * 16, 16)]], w16)
            return 0

        lax.fori_loop(0, ET // CH, chunk, 0)
        pltpu.sync_copy(wa, w_h.at[pl.ds(tbase, ET)])
        pltpu.sync_copy(dacc, dp_h.at[wid])

    return run(xl, xr, src, dst, att4)


def _phase_a(xle, xlo, src, dst3, w, zeros_np):
    mesh = plsc.VectorSubcoreMesh(core_axis_name="c", subcore_axis_name="s")
    NCHA = ETA // CHA

    @functools.partial(
        pl.kernel,
        out_type=jax.ShapeDtypeStruct((4, NP, 128), _f32),
        mesh=mesh,
        scratch_types=[
            pltpu.VMEM((2, CHA), jnp.int32),
            pltpu.VMEM((NCHA, 1, CHA), jnp.int32),
            pltpu.VMEM((2 * CHA,), _f32),
            pltpu.VMEM((2, CHA, 128), _f32),
            pltpu.VMEM_SHARED((NP, 128), _f32),
            pltpu.SemaphoreType.DMA,
            pltpu.SemaphoreType.DMA,
        ],
        compiler_params=pltpu.CompilerParams(needs_layout_passes=False),
    )
    def run(xle_h, xlo_h, src_h, dst3_h, w_h, z_h,
            num4_h, sib, di3, wbf, rows, acc, sem, sem2):
        c = lax.axis_index("c")
        s = lax.axis_index("s")
        ebase = s * ETA
        pltpu.sync_copy(dst3_h.at[pl.ds(s * NCHA, NCHA)], di3)

        def run_fb(tab, p, fb):
            xl_t = tab.at[p]
            num_t = num4_h.at[fb]
            pltpu.sync_copy(z_h.at[pl.ds(s * RT, RT)],
                            acc.at[pl.ds(s * RT, RT)])
            pltpu.sync_copy(src_h.at[pl.ds(ebase, CHA)], sib.at[0])
            pltpu.sync_copy(src_h.at[pl.ds(ebase + CHA, CHA)], sib.at[1])
            plsc.subcore_barrier()
            pltpu.async_copy(xl_t.at[sib.at[0]], rows.at[0], sem)
            pltpu.async_copy(w_h.at[pl.ds(ebase, CHA)],
                             wbf.at[pl.ds(0, CHA)], sem)

            def chunk(t, _):
                buf = lax.rem(t, 2)
                pltpu.make_async_copy(xl_t.at[pl.ds(0, CHA)],
                                      rows.at[buf], sem).wait()
                pltpu.make_async_copy(w_h.at[pl.ds(0, CHA)],
                                      wbf.at[pl.ds(buf * CHA, CHA)],
                                      sem).wait()

                @pl.when((t >= 1) & (t + 1 < NCHA))
                def _():
                    pltpu.make_async_copy(
                        src_h.at[pl.ds(0, CHA)],
                        sib.at[1 - buf], sem).wait()

                @pl.when(t >= 1)
                def _():
                    pltpu.make_async_copy(rows.at[1 - buf],
                                          acc.at[pl.ds(0, CHA)],
                                          sem2).wait()

                @pl.when(t + 1 < NCHA)
                def _():
                    pltpu.async_copy(xl_t.at[sib.at[1 - buf]],
                                     rows.at[1 - buf], sem)
                    pltpu.async_copy(
                        w_h.at[pl.ds(ebase + (t + 1) * CHA, CHA)],
                        wbf.at[pl.ds((1 - buf) * CHA, CHA)], sem)

                @pl.when(t + 2 < NCHA)
                def _():
                    pltpu.async_copy(
                        src_h.at[pl.ds(ebase + (t + 2) * CHA, CHA)],
                        sib.at[buf], sem)

                def edge(e, _):
                    wsp = plsc.load_gather(
                        wbf, [jnp.full((16,), buf * CHA + e, jnp.int32)])
                    for j in range(8):
                        sl = pl.ds(16 * j, 16)
                        rows[buf, e, sl] = rows[buf, e, sl] * wsp
                    return 0

                lax.fori_loop(0, CHA, edge, 0, unroll=4)
                pltpu.async_copy(rows.at[buf], acc.at[di3.at[t, 0]], sem2,
                                 add=True)
                return 0

            lax.fori_loop(0, NCHA, chunk, 0)
            pltpu.make_async_copy(rows.at[(NCHA - 1) % 2],
                                  acc.at[pl.ds(0, CHA)], sem2).wait()
            plsc.subcore_barrier()
            pltpu.sync_copy(acc.at[pl.ds(s * RT, RT)],
                            num_t.at[pl.ds(s * RT, RT)])
            plsc.subcore_barrier()

        @pl.when(c == 0)
        def _():
            run_fb(xle_h, 0, 0)
            run_fb(xlo_h, 0, 1)

        @pl.when(c == 1)
        def _():
            run_fb(xle_h, 1, 2)
            run_fb(xlo_h, 1, 3)

    return run(xle, xlo, src, dst3, w, zeros_np)


def kernel(x, edge_index, batch, Ws0, Wd0, att0, b0, Ws1, Wd1, att1, b1,
           Ws2, Wd2, att2, b2, W1, bm1, W2, bm2):
    loop = jnp.arange(N_NODES, dtype=jnp.int32)
    pad_e = E2 - E_LOOP
    src = jnp.concatenate([edge_index[0].astype(jnp.int32), loop,
                           jnp.zeros((pad_e,), jnp.int32)])
    dst = jnp.concatenate([edge_index[1].astype(jnp.int32), loop,
                           jnp.full((pad_e,), NP - 1, jnp.int32)])
    xp = jnp.pad(x, ((0, NP - N_NODES), (0, 0)))
    batch_p = jnp.pad(batch.astype(jnp.int32), (0, NP - N_NODES),
                      constant_values=G)
    zeros_np = jnp.zeros((NP, 128), _f32)

    dst3 = dst.reshape(E2 // CHA, 1, CHA)
    h4 = jnp.transpose(xp.reshape(NP, 2, 128), (1, 0, 2))
    for Ws, Wd, att, b in ((Ws0, Wd0, att0, b0), (Ws1, Wd1, att1, b1),
                           (Ws2, Wd2, att2, b2)):
        xle, xlo, xlp, xrp = _dual_mm(h4, Ws, Wd)
        ab = att.astype(jnp.bfloat16).reshape(2, 2, 128)
        alo = lax.bitcast_convert_type(
            ab[:, 0, :], jnp.uint16).astype(jnp.uint32)
        ahi = lax.bitcast_convert_type(
            ab[:, 1, :], jnp.uint16).astype(jnp.uint32)
        attp = lax.bitcast_convert_type(
            alo | (ahi << 16), jnp.int32).reshape(256)
        w, dp32 = _phase_e(xlp, xrp, src, dst, attp)
        num4 = _phase_a(xle, xlo, src, dst3, w, zeros_np)
        h4 = _normalize(num4, dp32, b.reshape(4, 1, 128))

    return _pool_mlp(h4, batch_p, W1, bm1.reshape(1, 512), W2,
                     bm2.reshape(1, 256))


# speedup vs baseline: 1.0197x; 1.0197x over previous
import functools

import jax
import jax.numpy as jnp
from jax import lax
from jax.experimental import pallas as pl
from jax.experimental.pallas import tpu as pltpu
from jax.experimental.pallas import tpu_sc as plsc

N_NODES = 10000
NP = 10240
G = 128
E_RAW = 160000
E_LOOP = E_RAW + N_NODES
CH = 64
CHA = 128
E2 = 172032
NC, NS = 2, 16
EH = E2 // NC
ET = EH // NS
ETA = E2 // NS
RT = NP // NS
NB = 512
NI = NP // NB

_f32 = jnp.float32


def _pack_i32(x):
    lo = lax.bitcast_convert_type(
        x[:, :128].astype(jnp.bfloat16), jnp.uint16).astype(jnp.uint32)
    hi = lax.bitcast_convert_type(
        x[:, 128:].astype(jnp.bfloat16), jnp.uint16).astype(jnp.uint32)
    return lax.bitcast_convert_type(lo | (hi << 16), jnp.int32)


def _dual_mm(h4, Ws, Wd):
    KB = h4.shape[0]

    def body(a_ref, ws_ref, wd_ref, xle_ref, xlo_ref, xlp_ref, xrp_ref,
             acc_l, acc_r):
        k = pl.program_id(2)

        @pl.when(k == 0)
        def _():
            acc_l[...] = jnp.zeros_like(acc_l)
            acc_r[...] = jnp.zeros_like(acc_r)

        a = a_ref[0]
        acc_l[...] += jnp.dot(a, ws_ref[...], preferred_element_type=_f32)
        acc_r[...] += jnp.dot(a, wd_ref[...], preferred_element_type=_f32)

        @pl.when(k == KB - 1)
        def _():
            al = acc_l[...]
            xle_ref[0] = al[:, :128]
            xlo_ref[0] = al[:, 128:]
            xlp_ref[0] = _pack_i32(al)
            xrp_ref[0] = _pack_i32(acc_r[...])

    two = jax.ShapeDtypeStruct((2, NP, 128), _f32)
    twoi = jax.ShapeDtypeStruct((2, NP, 128), jnp.int32)
    return pl.pallas_call(
        body,
        grid=(NI, 2, KB),
        in_specs=[
            pl.BlockSpec((1, NB, 128), lambda i, j, k: (k, i, 0)),
            pl.BlockSpec((128, 256), lambda i, j, k: (k, j)),
            pl.BlockSpec((128, 256), lambda i, j, k: (k, j)),
        ],
        out_specs=[
            pl.BlockSpec((1, NB, 128), lambda i, j, k: (j, i, 0)),
            pl.BlockSpec((1, NB, 128), lambda i, j, k: (j, i, 0)),
            pl.BlockSpec((1, NB, 128), lambda i, j, k: (j, i, 0)),
            pl.BlockSpec((1, NB, 128), lambda i, j, k: (j, i, 0)),
        ],
        out_shape=[two, two, twoi, twoi],
        scratch_shapes=[pltpu.VMEM((NB, 256), _f32),
                        pltpu.VMEM((NB, 256), _f32)],
    )(h4, Ws, Wd)


def _dual_mm_fused(num4, dp32, b4, Ws, Wd):
    def body(num_ref, dp_ref, b_ref, ws_ref, wd_ref,
             xle_ref, xlo_ref, xlp_ref, xrp_ref, acc_l, acc_r):
        k = pl.program_id(2)

        @pl.when(k == 0)
        def _():
            acc_l[...] = jnp.zeros_like(acc_l)
            acc_r[...] = jnp.zeros_like(acc_r)

        denom = jnp.sum(dp_ref[...], axis=0) + 1e-16
        a = jnp.maximum(num_ref[0] / denom[:, None] + b_ref[0], 0.0)
        acc_l[...] += jnp.dot(a, ws_ref[...], preferred_element_type=_f32)
        acc_r[...] += jnp.dot(a, wd_ref[...], preferred_element_type=_f32)

        @pl.when(k == 3)
        def _():
            al = acc_l[...]
            xle_ref[0] = al[:, :128]
            xlo_ref[0] = al[:, 128:]
            xlp_ref[0] = _pack_i32(al)
            xrp_ref[0] = _pack_i32(acc_r[...])

    two = jax.ShapeDtypeStruct((2, NP, 128), _f32)
    twoi = jax.ShapeDtypeStruct((2, NP, 128), jnp.int32)
    return pl.pallas_call(
        body,
        grid=(NI, 2, 4),
        in_specs=[
            pl.BlockSpec((1, NB, 128), lambda i, j, k: (k, i, 0)),
            pl.BlockSpec((32, NB), lambda i, j, k: (0, i)),
            pl.BlockSpec((1, 1, 128), lambda i, j, k: (k, 0, 0)),
            pl.BlockSpec((128, 256), lambda i, j, k: (k, j)),
            pl.BlockSpec((128, 256), lambda i, j, k: (k, j)),
        ],
        out_specs=[
            pl.BlockSpec((1, NB, 128), lambda i, j, k: (j, i, 0)),
            pl.BlockSpec((1, NB, 128), lambda i, j, k: (j, i, 0)),
            pl.BlockSpec((1, NB, 128), lambda i, j, k: (j, i, 0)),
            pl.BlockSpec((1, NB, 128), lambda i, j, k: (j, i, 0)),
        ],
        out_shape=[two, two, twoi, twoi],
        scratch_shapes=[pltpu.VMEM((NB, 256), _f32),
                        pltpu.VMEM((NB, 256), _f32)],
    )(num4, dp32, b4, Ws, Wd)


def _normalize(num4, dp32, b4):
    def body(num_ref, dp_ref, b_ref, out_ref):
        denom = jnp.sum(dp_ref[...], axis=0) + 1e-16
        h = num_ref[0] / denom[:, None] + b_ref[0]
        out_ref[0] = jnp.maximum(h, 0.0)

    return pl.pallas_call(
        body,
        grid=(4, NI),
        in_specs=[
            pl.BlockSpec((1, NB, 128), lambda kb, i: (kb, i, 0)),
            pl.BlockSpec((32, NB), lambda kb, i: (0, i)),
            pl.BlockSpec((1, 1, 128), lambda kb, i: (kb, 0, 0)),
        ],
        out_specs=pl.BlockSpec((1, NB, 128), lambda kb, i: (kb, i, 0)),
        out_shape=jax.ShapeDtypeStruct((4, NP, 128), _f32),
    )(num4, dp32, b4)


def _pool_mlp(h4, batch_p, W1, bm1, W2, bm2):
    def body(h_ref, bt_ref, w1_ref, b1_ref, w2_ref, b2_ref, out_ref,
             sums_ref, cnt_ref):
        i = pl.program_id(0)

        @pl.when(i == 0)
        def _():
            sums_ref[...] = jnp.zeros_like(sums_ref)
            cnt_ref[...] = jnp.zeros_like(cnt_ref)

        gid = lax.broadcasted_iota(jnp.int32, (G, NB), 0)
        oh = (gid == bt_ref[...][None, :]).astype(_f32)
        for kb in range(4):
            sums_ref[:, kb * 128:(kb + 1) * 128] += jnp.dot(
                oh, h_ref[kb], preferred_element_type=_f32)
        cnt_ref[...] += jnp.dot(oh, jnp.ones((NB, 128), _f32),
                                preferred_element_type=_f32)

        @pl.when(i == NI - 1)
        def _():
            cnt = jnp.maximum(cnt_ref[:, 0:1], 1.0)
            pooled = sums_ref[...] / cnt
            z = jnp.maximum(
                jnp.dot(pooled, w1_ref[...], preferred_element_type=_f32)
                + b1_ref[...], 0.0)
            out_ref[...] = (jnp.dot(z, w2_ref[...],
                                    preferred_element_type=_f32)
                            + b2_ref[...])

    return pl.pallas_call(
        body,
        grid=(NI,),
        in_specs=[
            pl.BlockSpec((4, NB, 128), lambda i: (0, i, 0)),
            pl.BlockSpec((NB,), lambda i: (i,)),
            pl.BlockSpec((512, 512), lambda i: (0, 0)),
            pl.BlockSpec((1, 512), lambda i: (0, 0)),
            pl.BlockSpec((512, 256), lambda i: (0, 0)),
            pl.BlockSpec((1, 256), lambda i: (0, 0)),
        ],
        out_specs=pl.BlockSpec((G, 256), lambda i: (0, 0)),
        out_shape=jax.ShapeDtypeStruct((G, 256), _f32),
        scratch_shapes=[
            pltpu.VMEM((G, 512), _f32),
            pltpu.VMEM((G, 128), _f32),
        ],
    )(h4, batch_p, W1, bm1, W2, bm2)


def _phase_e(xl, xr, src, dst, att4):
    mesh = plsc.VectorSubcoreMesh(core_axis_name="c", subcore_axis_name="s")
    bf16 = jnp.bfloat16

    @functools.partial(
        pl.kernel,
        out_type=(jax.ShapeDtypeStruct((E2,), _f32),
                  jax.ShapeDtypeStruct((NC * NS, NP), _f32)),
        mesh=mesh,
        scratch_types=[
            pltpu.VMEM((ET,), jnp.int32),
            pltpu.VMEM((ET,), jnp.int32),
            pltpu.VMEM((2, 2, CH, 128), jnp.int32),
            pltpu.VMEM((2, 2, CH, 128), jnp.int32),
            pltpu.VMEM((ET,), _f32),
            pltpu.VMEM((CH * 16,), _f32),
            pltpu.VMEM((NP,), _f32),
            pltpu.VMEM((256,), jnp.int32),
            pltpu.SemaphoreType.DMA,
        ],
        compiler_params=pltpu.CompilerParams(needs_layout_passes=False),
    )
    def run(xlp_h, xrp_h, src_h, dst_h, att_h,
            w_h, dp_h, si, di, xlb, xrb, wa, tmpb, dacc, attb, sem):
        c = lax.axis_index("c")
        s = lax.axis_index("s")
        wid = c * NS + s
        tbase = c * EH + s * ET
        pltpu.sync_copy(src_h.at[pl.ds(tbase, ET)], si)
        pltpu.sync_copy(dst_h.at[pl.ds(tbase, ET)], di)
        pltpu.sync_copy(att_h, attb)

        def zero(i, _):
            dacc[pl.ds(i * 16, 16)] = jnp.zeros((16,), _f32)
            return 0

        lax.fori_loop(0, NP // 16, zero, 0)
        att_regs = [plsc.bitcast(attb[pl.ds(128 * tb + 16 * j, 16)], bf16)
                    for tb in range(2) for j in range(8)]

        def fire(t, buf):
            isl = si.at[pl.ds(t * CH, CH)]
            idl = di.at[pl.ds(t * CH, CH)]
            for tb in range(2):
                pltpu.async_copy(xlp_h.at[tb].at[isl], xlb.at[buf, tb], sem)
            for tb in range(2):
                pltpu.async_copy(xrp_h.at[tb].at[idl], xrb.at[buf, tb], sem)

        def drain(buf):
            for tb in range(2):
                pltpu.make_async_copy(xlp_h.at[tb].at[pl.ds(0, CH)],
                                      xlb.at[buf, tb], sem).wait()
            for tb in range(2):
                pltpu.make_async_copy(xrp_h.at[tb].at[pl.ds(0, CH)],
                                      xrb.at[buf, tb], sem).wait()

        fire(0, 0)

        def chunk(t, _):
            buf = lax.rem(t, 2)
            drain(buf)

            @pl.when(t + 1 < ET // CH)
            def _():
                fire(t + 1, 1 - buf)

            def edge(e, _):
                acc = jnp.zeros((16,), _f32)
                for tb in range(2):
                    for j in range(8):
                        a = plsc.bitcast(
                            xlb[buf, tb, e, pl.ds(16 * j, 16)], bf16)
                        bb = plsc.bitcast(
                            xrb[buf, tb, e, pl.ds(16 * j, 16)], bf16)
                        h = a + bb
                        r = jnp.maximum(h, 0) + bf16(0.2) * jnp.minimum(h, 0)
                        m = r * att_regs[tb * 8 + j]
                        u0, u1 = plsc.unpack(
                            m, format=plsc.PackFormat.INTERLEAVED)
                        acc = acc + (u0 + u1)
                tmpb[pl.ds(e * 16, 16)] = acc
                return 0

            lax.fori_loop(0, CH, edge, 0, unroll=4)
            lane = lax.broadcasted_iota(jnp.int32, (16,), 0)
            for q in range(CH // 16):
                rowi = (q * 16 + lane) * 16
                sv = jnp.zeros((16,), _f32)
                for k in range(16):
                    sv = sv + plsc.load_gather(tmpb, [rowi + k])
                w16 = jnp.exp(sv)
                wa[pl.ds(t * CH + q * 16, 16)] = w16
                plsc.addupdate_scatter(
                    dacc, [di[pl.ds(t * CH + q * 16, 16)]], w16)
            return 0

        lax.fori_loop(0, ET // CH, chunk, 0)
        pltpu.sync_copy(wa, w_h.at[pl.ds(tbase, ET)])
        pltpu.sync_copy(dacc, dp_h.at[wid])

    return run(xl, xr, src, dst, att4)


def _phase_a(xle, xlo, src, dst3, w, zeros_np):
    mesh = plsc.VectorSubcoreMesh(core_axis_name="c", subcore_axis_name="s")
    NCHA = ETA // CHA

    @functools.partial(
        pl.kernel,
        out_type=jax.ShapeDtypeStruct((4, NP, 128), _f32),
        mesh=mesh,
        scratch_types=[
            pltpu.VMEM((2, CHA), jnp.int32),
            pltpu.VMEM((NCHA, 1, CHA), jnp.int32),
            pltpu.VMEM((2 * CHA,), _f32),
            pltpu.VMEM((2, CHA, 128), _f32),
            pltpu.VMEM_SHARED((NP, 128), _f32),
            pltpu.SemaphoreType.DMA,
            pltpu.SemaphoreType.DMA,
        ],
        compiler_params=pltpu.CompilerParams(needs_layout_passes=False),
    )
    def run(xle_h, xlo_h, src_h, dst3_h, w_h, z_h,
            num4_h, sib, di3, wbf, rows, acc, sem, sem2):
        c = lax.axis_index("c")
        s = lax.axis_index("s")
        ebase = s * ETA
        pltpu.sync_copy(dst3_h.at[pl.ds(s * NCHA, NCHA)], di3)

        def run_fb(tab, p, fb):
            xl_t = tab.at[p]
            num_t = num4_h.at[fb]
            pltpu.sync_copy(z_h.at[pl.ds(s * RT, RT)],
                            acc.at[pl.ds(s * RT, RT)])
            pltpu.sync_copy(src_h.at[pl.ds(ebase, CHA)], sib.at[0])
            pltpu.sync_copy(src_h.at[pl.ds(ebase + CHA, CHA)], sib.at[1])
            plsc.subcore_barrier()
            pltpu.async_copy(xl_t.at[sib.at[0]], rows.at[0], sem)
            pltpu.async_copy(w_h.at[pl.ds(ebase, CHA)],
                             wbf.at[pl.ds(0, CHA)], sem)

            def chunk(t, _):
                buf = lax.rem(t, 2)
                pltpu.make_async_copy(xl_t.at[pl.ds(0, CHA)],
                                      rows.at[buf], sem).wait()
                pltpu.make_async_copy(w_h.at[pl.ds(0, CHA)],
                                      wbf.at[pl.ds(buf * CHA, CHA)],
                                      sem).wait()

                @pl.when((t >= 1) & (t + 1 < NCHA))
                def _():
                    pltpu.make_async_copy(
                        src_h.at[pl.ds(0, CHA)],
                        sib.at[1 - buf], sem).wait()

                @pl.when(t >= 1)
                def _():
                    pltpu.make_async_copy(rows.at[1 - buf],
                                          acc.at[pl.ds(0, CHA)],
                                          sem2).wait()

                @pl.when(t + 1 < NCHA)
                def _():
                    pltpu.async_copy(xl_t.at[sib.at[1 - buf]],
                                     rows.at[1 - buf], sem)
                    pltpu.async_copy(
                        w_h.at[pl.ds(ebase + (t + 1) * CHA, CHA)],
                        wbf.at[pl.ds((1 - buf) * CHA, CHA)], sem)

                @pl.when(t + 2 < NCHA)
                def _():
                    pltpu.async_copy(
                        src_h.at[pl.ds(ebase + (t + 2) * CHA, CHA)],
                        sib.at[buf], sem)

                def edge(e, _):
                    wsp = plsc.load_gather(
                        wbf, [jnp.full((16,), buf * CHA + e, jnp.int32)])
                    for j in range(8):
                        sl = pl.ds(16 * j, 16)
                        rows[buf, e, sl] = rows[buf, e, sl] * wsp
                    return 0

                lax.fori_loop(0, CHA, edge, 0, unroll=4)
                pltpu.async_copy(rows.at[buf], acc.at[di3.at[t, 0]], sem2,
                                 add=True)
                return 0

            lax.fori_loop(0, NCHA, chunk, 0)
            pltpu.make_async_copy(rows.at[(NCHA - 1) % 2],
                                  acc.at[pl.ds(0, CHA)], sem2).wait()
            plsc.subcore_barrier()
            pltpu.sync_copy(acc.at[pl.ds(s * RT, RT)],
                            num_t.at[pl.ds(s * RT, RT)])
            plsc.subcore_barrier()

        @pl.when(c == 0)
        def _():
            run_fb(xle_h, 0, 0)
            run_fb(xlo_h, 0, 1)

        @pl.when(c == 1)
        def _():
            run_fb(xle_h, 1, 2)
            run_fb(xlo_h, 1, 3)

    return run(xle, xlo, src, dst3, w, zeros_np)


def kernel(x, edge_index, batch, Ws0, Wd0, att0, b0, Ws1, Wd1, att1, b1,
           Ws2, Wd2, att2, b2, W1, bm1, W2, bm2):
    loop = jnp.arange(N_NODES, dtype=jnp.int32)
    pad_e = E2 - E_LOOP
    src = jnp.concatenate([edge_index[0].astype(jnp.int32), loop,
                           jnp.zeros((pad_e,), jnp.int32)])
    dst = jnp.concatenate([edge_index[1].astype(jnp.int32), loop,
                           jnp.full((pad_e,), NP - 1, jnp.int32)])
    xp = jnp.pad(x, ((0, NP - N_NODES), (0, 0)))
    batch_p = jnp.pad(batch.astype(jnp.int32), (0, NP - N_NODES),
                      constant_values=G)
    zeros_np = jnp.zeros((NP, 128), _f32)

    dst3 = dst.reshape(E2 // CHA, 1, CHA)
    h4 = jnp.transpose(xp.reshape(NP, 2, 128), (1, 0, 2))
    num4 = dp32_prev = b_prev = None
    for Ws, Wd, att, b in ((Ws0, Wd0, att0, b0), (Ws1, Wd1, att1, b1),
                           (Ws2, Wd2, att2, b2)):
        if num4 is None:
            xle, xlo, xlp, xrp = _dual_mm(h4, Ws, Wd)
        else:
            xle, xlo, xlp, xrp = _dual_mm_fused(
                num4, dp32_prev, b_prev.reshape(4, 1, 128), Ws, Wd)
        ab = att.astype(jnp.bfloat16).reshape(2, 2, 128)
        alo = lax.bitcast_convert_type(
            ab[:, 0, :], jnp.uint16).astype(jnp.uint32)
        ahi = lax.bitcast_convert_type(
            ab[:, 1, :], jnp.uint16).astype(jnp.uint32)
        attp = lax.bitcast_convert_type(
            alo | (ahi << 16), jnp.int32).reshape(256)
        w, dp32 = _phase_e(xlp, xrp, src, dst, attp)
        num4 = _phase_a(xle, xlo, src, dst3, w, zeros_np)
        dp32_prev, b_prev = dp32, b

    h4 = _normalize(num4, dp32_prev, b_prev.reshape(4, 1, 128))
    return _pool_mlp(h4, batch_p, W1, bm1.reshape(1, 512), W2,
                     bm2.reshape(1, 256))
